# Initial kernel scaffold; baseline (speedup 1.0000x reference)
import jax
import jax.numpy as jnp
from jax.experimental import pallas as pl


def kernel(x, edge_index, W1, b1, W2, b2, W3, b3):
    raise NotImplementedError("write your pallas kernel here")


# speedup vs baseline: 12.0883x; 12.0883x over previous
import functools

import jax
import jax.numpy as jnp
from jax import lax
from jax.experimental import pallas as pl
from jax.experimental.pallas import tpu as pltpu
from jax.experimental.pallas import tpu_sc as plsc

N = 10000
E = 160000
NC, NS = 2, 16
NW = NC * NS
N_PAD = 10240
EPC = E // NC
EPW = E // NW
C = 128
NFULL = EPW // C
REM = EPW - NFULL * C
RPT = N_PAD // NS
ZR = 40

_mesh = lambda: plsc.VectorSubcoreMesh(core_axis_name="c", subcore_axis_name="s")
_SC_PARAMS = pltpu.CompilerParams(use_tc_tiling_on_sc=False)


@functools.partial(
    pl.kernel,
    out_type=jax.ShapeDtypeStruct((NC * N_PAD,), jnp.float32),
    mesh=_mesh(),
    scratch_types=[
        pltpu.VMEM_SHARED((N_PAD,), jnp.float32),
        pltpu.VMEM((C,), jnp.int32),
        pltpu.VMEM((C,), jnp.float32),
        pltpu.VMEM((RPT,), jnp.float32),
        pltpu.SemaphoreType.DMA,
    ],
    compiler_params=_SC_PARAMS,
)
def _deg(dst_hbm, out_hbm, acc, didx, ones_v, zbuf, sem):
    cid = lax.axis_index("c")
    sid = lax.axis_index("s")
    for i in range(C // 16):
        ones_v[pl.ds(i * 16, 16)] = jnp.full((16,), 1.0, jnp.float32)
    for i in range(RPT // 16):
        zbuf[pl.ds(i * 16, 16)] = jnp.zeros((16,), jnp.float32)
    pltpu.sync_copy(zbuf, acc.at[pl.ds(sid * RPT, RPT)])
    plsc.subcore_barrier()

    base = cid * EPC + sid * EPW

    @pl.loop(0, NFULL)
    def _(j):
        pltpu.sync_copy(dst_hbm.at[pl.ds(base + j * C, C)], didx)
        pltpu.sync_copy(ones_v, acc.at[didx], add=True)

    didx_r = didx.at[pl.ds(0, REM)]
    pltpu.sync_copy(dst_hbm.at[pl.ds(base + NFULL * C, REM)], didx_r)
    pltpu.sync_copy(ones_v.at[pl.ds(0, REM)], acc.at[didx_r], add=True)

    plsc.subcore_barrier()
    pltpu.sync_copy(acc.at[pl.ds(sid * RPT, RPT)],
                    out_hbm.at[pl.ds(cid * N_PAD + sid * RPT, RPT)])


def _make_prop(F):
    @functools.partial(
        pl.kernel,
        out_type=jax.ShapeDtypeStruct((NC * N_PAD, F), jnp.float32),
        mesh=_mesh(),
        scratch_types=[
            pltpu.VMEM_SHARED((N_PAD, F), jnp.float32),
            pltpu.VMEM((C,), jnp.int32),
            pltpu.VMEM((C,), jnp.int32),
            pltpu.VMEM((C, F), jnp.float32),
            pltpu.VMEM((ZR, F), jnp.float32),
            pltpu.SemaphoreType.DMA,
        ],
        compiler_params=_SC_PARAMS,
    )
    def prop(g_hbm, src_hbm, dst_hbm, out_hbm, acc, sidx, didx, rows, zbuf, sem):
        cid = lax.axis_index("c")
        sid = lax.axis_index("s")
        for r in range(ZR):
            for q in range(F // 16):
                zbuf[r, pl.ds(q * 16, 16)] = jnp.zeros((16,), jnp.float32)
        for k in range(RPT // ZR):
            pltpu.sync_copy(zbuf, acc.at[pl.ds(sid * RPT + k * ZR, ZR)])
        plsc.subcore_barrier()

        base = cid * EPC + sid * EPW

        @pl.loop(0, NFULL)
        def _(j):
            pltpu.sync_copy(src_hbm.at[pl.ds(base + j * C, C)], sidx)
            pltpu.sync_copy(dst_hbm.at[pl.ds(base + j * C, C)], didx)
            pltpu.async_copy(g_hbm.at[sidx], rows, sem).wait()
            pltpu.sync_copy(rows, acc.at[didx], add=True)

        sidx_r = sidx.at[pl.ds(0, REM)]
        didx_r = didx.at[pl.ds(0, REM)]
        rows_r = rows.at[pl.ds(0, REM)]
        pltpu.sync_copy(src_hbm.at[pl.ds(base + NFULL * C, REM)], sidx_r)
        pltpu.sync_copy(dst_hbm.at[pl.ds(base + NFULL * C, REM)], didx_r)
        pltpu.async_copy(g_hbm.at[sidx_r], rows_r, sem).wait()
        pltpu.sync_copy(rows_r, acc.at[didx_r], add=True)

        plsc.subcore_barrier()
        pltpu.sync_copy(acc.at[pl.ds(sid * RPT, RPT)],
                        out_hbm.at[pl.ds(cid * N_PAD + sid * RPT, RPT)])

    return prop


_prop128 = _make_prop(128)
_prop64 = _make_prop(64)


R = 1000


def _tc_first(x, W, c0, c1):
    Din, Dout = W.shape

    def body(x_ref, w_ref, c0_ref, c1_ref, g_ref, dinv_ref):
        h = jnp.dot(x_ref[...], w_ref[...], preferred_element_type=jnp.float32)
        dinv = lax.rsqrt(c0_ref[...] + c1_ref[...] + 1.0)
        g_ref[...] = h * dinv
        dinv_ref[...] = dinv

    return pl.pallas_call(
        body,
        grid=(N // R,),
        in_specs=[
            pl.BlockSpec((R, Din), lambda i: (i, 0)),
            pl.BlockSpec((Din, Dout), lambda i: (0, 0)),
            pl.BlockSpec((R, 1), lambda i: (i, 0)),
            pl.BlockSpec((R, 1), lambda i: (i, 0)),
        ],
        out_specs=[
            pl.BlockSpec((R, Dout), lambda i: (i, 0)),
            pl.BlockSpec((R, 1), lambda i: (i, 0)),
        ],
        out_shape=[
            jax.ShapeDtypeStruct((N, Dout), jnp.float32),
            jax.ShapeDtypeStruct((N, 1), jnp.float32),
        ],
    )(x, W, c0, c1)


def _tc_mid(a0, a1, g, dinv, b, W):
    Din, Dout = W.shape

    def body(a0_ref, a1_ref, g_ref, dinv_ref, b_ref, w_ref, o_ref):
        s = dinv_ref[...] * (a0_ref[...] + a1_ref[...] + g_ref[...]) + b_ref[...]
        act = jnp.where(s >= 0, s, 0.2 * s)
        h = jnp.dot(act, w_ref[...], preferred_element_type=jnp.float32)
        o_ref[...] = h * dinv_ref[...]

    return pl.pallas_call(
        body,
        grid=(N // R,),
        in_specs=[
            pl.BlockSpec((R, Din), lambda i: (i, 0)),
            pl.BlockSpec((R, Din), lambda i: (i, 0)),
            pl.BlockSpec((R, Din), lambda i: (i, 0)),
            pl.BlockSpec((R, 1), lambda i: (i, 0)),
            pl.BlockSpec((1, Din), lambda i: (0, 0)),
            pl.BlockSpec((Din, Dout), lambda i: (0, 0)),
        ],
        out_specs=pl.BlockSpec((R, Dout), lambda i: (i, 0)),
        out_shape=jax.ShapeDtypeStruct((N, Dout), jnp.float32),
    )(a0, a1, g, dinv, b, W)


def _tc_last(a0, a1, g, dinv, b):
    F = g.shape[1]

    def body(a0_ref, a1_ref, g_ref, dinv_ref, b_ref, o_ref):
        o_ref[...] = (dinv_ref[...] * (a0_ref[...] + a1_ref[...] + g_ref[...])
                      + b_ref[...])

    return pl.pallas_call(
        body,
        grid=(N // R,),
        in_specs=[
            pl.BlockSpec((R, F), lambda i: (i, 0)),
            pl.BlockSpec((R, F), lambda i: (i, 0)),
            pl.BlockSpec((R, F), lambda i: (i, 0)),
            pl.BlockSpec((R, 1), lambda i: (i, 0)),
            pl.BlockSpec((1, F), lambda i: (0, 0)),
        ],
        out_specs=pl.BlockSpec((R, F), lambda i: (i, 0)),
        out_shape=jax.ShapeDtypeStruct((N, F), jnp.float32),
    )(a0, a1, g, dinv, b)


def kernel(x, edge_index, W1, b1, W2, b2, W3, b3):
    ei = edge_index.astype(jnp.int32)
    src, dst = ei[0], ei[1]

    cnt = _deg(dst)
    c0 = cnt[:N].reshape(N, 1)
    c1 = cnt[N_PAD:N_PAD + N].reshape(N, 1)

    g1, dinv = _tc_first(x, W1, c0, c1)
    acc = _prop128(g1, src, dst)
    g2 = _tc_mid(acc[:N], acc[N_PAD:N_PAD + N], g1, dinv, b1.reshape(1, -1), W2)
    acc = _prop64(g2, src, dst)
    g3 = _tc_mid(acc[:N], acc[N_PAD:N_PAD + N], g2, dinv, b2.reshape(1, -1), W3)
    acc = _prop64(g3, src, dst)
    return _tc_last(acc[:N], acc[N_PAD:N_PAD + N], g3, dinv, b3.reshape(1, -1))
